# baseline (speedup 1.0000x reference)
import functools

import jax
import jax.numpy as jnp
from jax import lax
from jax.experimental import pallas as pl
from jax.experimental.pallas import tpu as pltpu
from jax.experimental.pallas import tpu_sc as plsc

N = 1000000
B = 4096
W = 2 * B - 2
RHO = 0.99
ALPHA = 0.5
NSQ = float(N) ** 2
UPD_SCALE = (1.0 - RHO) * NSQ

ROWS_PER_BLOCK = 128
NUM_BLOCKS = B // ROWS_PER_BLOCK

NUM_TILES = 32
CHUNK = 31256
LAST_CHUNK = N - 31 * CHUNK
SCATTER_TILES = 16
UPD_PER_TILE = B // SCATTER_TILES


COPY_BLOCK = 31 * 1024
COPY_LAST = (N - 1) // COPY_BLOCK


def _tc_body(q1_ref, q2_ref, a1_ref, a2_ref, idxb_ref, idxf_ref, sinv_ref,
             xim_ref, last_ref, sa_ref, sr_ref, scopy_ref):
    i = pl.program_id(0)
    scopy_ref[...] = sinv_ref[...]
    rs1 = jnp.sum(q1_ref[...], axis=1)
    rs2 = jnp.sum(q2_ref[...], axis=1)
    rsum = rs1 + rs2
    xim_ref[...] = (0.5 * ALPHA) * (a1_ref[...] + a2_ref[...]) \
        + (0.5 * (1.0 - ALPHA) / W) * rsum

    eq = idxb_ref[...][:, None] == idxf_ref[...][None, :]
    jpos = lax.broadcasted_iota(jnp.int32, (ROWS_PER_BLOCK, B), 1)
    last_ref[...] = jnp.max(jnp.where(eq, jpos, -1), axis=1)

    @pl.when(i == 0)
    def _():
        sa_ref[...] = jnp.zeros_like(sa_ref)
        sr_ref[...] = jnp.zeros_like(sr_ref)

    sa_ref[...] = sa_ref[...] + (jnp.sum(a1_ref[...]) + jnp.sum(a2_ref[...]))
    sr_ref[...] = sr_ref[...] + jnp.sum(rsum)


_tc_call = pl.pallas_call(
    _tc_body,
    grid=(NUM_BLOCKS,),
    in_specs=[
        pl.BlockSpec((ROWS_PER_BLOCK, W), lambda i: (i, 0)),
        pl.BlockSpec((ROWS_PER_BLOCK, W), lambda i: (i, 0)),
        pl.BlockSpec((ROWS_PER_BLOCK,), lambda i: (i,)),
        pl.BlockSpec((ROWS_PER_BLOCK,), lambda i: (i,)),
        pl.BlockSpec((ROWS_PER_BLOCK,), lambda i: (i,)),
        pl.BlockSpec((B,), lambda i: (0,)),
        pl.BlockSpec((COPY_BLOCK,), lambda i: (jnp.minimum(i, COPY_LAST),)),
    ],
    out_specs=[
        pl.BlockSpec((ROWS_PER_BLOCK,), lambda i: (i,)),
        pl.BlockSpec((ROWS_PER_BLOCK,), lambda i: (i,)),
        pl.BlockSpec((1, 1), lambda i: (0, 0)),
        pl.BlockSpec((1, 1), lambda i: (0, 0)),
        pl.BlockSpec((COPY_BLOCK,), lambda i: (jnp.minimum(i, COPY_LAST),)),
    ],
    out_shape=[
        jax.ShapeDtypeStruct((B,), jnp.float32),
        jax.ShapeDtypeStruct((B,), jnp.int32),
        jax.ShapeDtypeStruct((1, 1), jnp.float32),
        jax.ShapeDtypeStruct((1, 1), jnp.float32),
        jax.ShapeDtypeStruct((N,), jnp.float32),
    ],
    compiler_params=pltpu.CompilerParams(
        dimension_semantics=("arbitrary",),
    ),
)


@functools.cache
def _make_sc_copy():
    return functools.partial(
        pl.kernel,
        mesh=plsc.VectorSubcoreMesh(core_axis_name="c", subcore_axis_name="s"),
        out_type=jax.ShapeDtypeStruct((N,), jnp.float32),
        scratch_types=[
            pltpu.VMEM((CHUNK,), jnp.float32),
            pltpu.SemaphoreType.DMA,
        ],
        compiler_params=pltpu.CompilerParams(needs_layout_passes=False),
    )(_sc_copy_body)


def _sc_copy_body(s_inv_hbm, out_hbm, chunk_v, sem):
    wid = lax.axis_index("s") * 2 + lax.axis_index("c")
    base = pl.multiple_of(wid * CHUNK, 8)
    is_last = wid == NUM_TILES - 1

    @pl.when(jnp.logical_not(is_last))
    def _():
        pltpu.async_copy(s_inv_hbm.at[pl.ds(base, CHUNK)], chunk_v,
                         sem).wait()
        pltpu.sync_copy(chunk_v, out_hbm.at[pl.ds(base, CHUNK)])

    @pl.when(is_last)
    def _():
        pltpu.async_copy(s_inv_hbm.at[pl.ds(31 * CHUNK, LAST_CHUNK)],
                         chunk_v.at[pl.ds(0, LAST_CHUNK)], sem).wait()
        pltpu.sync_copy(chunk_v.at[pl.ds(0, LAST_CHUNK)],
                        out_hbm.at[pl.ds(31 * CHUNK, LAST_CHUNK)])


@functools.cache
def _make_sc_scatter():
    return functools.partial(
        pl.kernel,
        mesh=plsc.VectorSubcoreMesh(core_axis_name="c", subcore_axis_name="s",
                                    num_cores=1),
        out_type=(),
        scratch_types=[
            pltpu.VMEM((UPD_PER_TILE,), jnp.int32),
            pltpu.VMEM((UPD_PER_TILE,), jnp.int32),
            pltpu.VMEM((UPD_PER_TILE,), jnp.float32),
            pltpu.VMEM((UPD_PER_TILE,), jnp.float32),
            pltpu.VMEM((UPD_PER_TILE,), jnp.float32),
            pltpu.SemaphoreType.DMA,
            pltpu.SemaphoreType.DMA,
            pltpu.SemaphoreType.DMA,
            pltpu.SemaphoreType.DMA,
            pltpu.SemaphoreType.DMA,
        ],
        compiler_params=pltpu.CompilerParams(needs_layout_passes=False),
    )(_sc_scatter_body)


def _sc_scatter_body(out_ref, s_inv_hbm, idx_hbm, xim_hbm, last_hbm,
                     idx_v, last_v, ximf_v, sold_v, vals_v,
                     sem_i, sem_l, sem_x, sem_s, sem_o):
    wid = lax.axis_index("s")
    pos = pl.multiple_of(wid * UPD_PER_TILE, 8)

    cp_i = pltpu.async_copy(idx_hbm.at[pl.ds(pos, UPD_PER_TILE)], idx_v,
                            sem_i)
    cp_l = pltpu.async_copy(last_hbm.at[pl.ds(pos, UPD_PER_TILE)], last_v,
                            sem_l)
    cp_i.wait()
    cp_s = pltpu.async_copy(s_inv_hbm.at[idx_v], sold_v, sem_s)
    cp_l.wait()
    cp_x = pltpu.async_copy(xim_hbm.at[last_v], ximf_v, sem_x)
    cp_s.wait()
    cp_x.wait()

    for k in range(UPD_PER_TILE // 16):
        sl = pl.ds(k * 16, 16)
        vals_v[sl] = RHO * sold_v[sl] + UPD_SCALE * ximf_v[sl]

    pltpu.async_copy(vals_v, out_ref.at[idx_v], sem_o).wait()


def kernel(q_attr_1, q_attr_2, q_rep_1, q_rep_2, feats_idx, s_inv,
           E_attr, E_rep):
    xim, last, sa, sr, buf = _tc_call(q_rep_1, q_rep_2, q_attr_1, q_attr_2,
                                      feats_idx, feats_idx, s_inv)
    ref = jax.new_ref(buf)
    _make_sc_scatter()(ref, s_inv, feats_idx, xim, last)
    s_inv_new = ref[...]
    w = NSQ / (NSQ + 2.0 * B * 100000.0)
    E_attr_new = (1.0 - w) * E_attr + (w / (2.0 * B)) * sa.reshape(1)
    E_rep_new = (1.0 - w) * E_rep + (w / (2.0 * B * W)) * sr.reshape(1)
    return (s_inv_new, E_attr_new, E_rep_new)


# speedup vs baseline: 1.0026x; 1.0026x over previous
import functools

import jax
import jax.numpy as jnp
from jax import lax
from jax.experimental import pallas as pl
from jax.experimental.pallas import tpu as pltpu
from jax.experimental.pallas import tpu_sc as plsc

N = 1000000
B = 4096
W = 2 * B - 2
RHO = 0.99
ALPHA = 0.5
NSQ = float(N) ** 2
UPD_SCALE = (1.0 - RHO) * NSQ

ROWS_PER_BLOCK = 128
NUM_BLOCKS = B // ROWS_PER_BLOCK

SCATTER_TILES = 16
UPD_PER_TILE = B // SCATTER_TILES


COPY_BLOCK = 31 * 1024
COPY_LAST = (N - 1) // COPY_BLOCK


def _tc_body(q1_ref, q2_ref, a1_ref, a2_ref, idxb_ref, idxf_ref, sinv_ref,
             xim_ref, last_ref, sa_ref, sr_ref, scopy_ref):
    i = pl.program_id(0)
    scopy_ref[...] = sinv_ref[...]
    rs1 = jnp.sum(q1_ref[...], axis=1)
    rs2 = jnp.sum(q2_ref[...], axis=1)
    rsum = rs1 + rs2
    xim_ref[...] = (0.5 * ALPHA) * (a1_ref[...] + a2_ref[...]) \
        + (0.5 * (1.0 - ALPHA) / W) * rsum

    eq = idxb_ref[...][:, None] == idxf_ref[...][None, :]
    jpos = lax.broadcasted_iota(jnp.int32, (ROWS_PER_BLOCK, B), 1)
    last_ref[...] = jnp.max(jnp.where(eq, jpos, -1), axis=1)

    @pl.when(i == 0)
    def _():
        sa_ref[...] = jnp.zeros_like(sa_ref)
        sr_ref[...] = jnp.zeros_like(sr_ref)

    sa_ref[...] = sa_ref[...] + (jnp.sum(a1_ref[...]) + jnp.sum(a2_ref[...]))
    sr_ref[...] = sr_ref[...] + jnp.sum(rsum)


_tc_call = pl.pallas_call(
    _tc_body,
    grid=(NUM_BLOCKS,),
    in_specs=[
        pl.BlockSpec((ROWS_PER_BLOCK, W), lambda i: (i, 0)),
        pl.BlockSpec((ROWS_PER_BLOCK, W), lambda i: (i, 0)),
        pl.BlockSpec((ROWS_PER_BLOCK,), lambda i: (i,)),
        pl.BlockSpec((ROWS_PER_BLOCK,), lambda i: (i,)),
        pl.BlockSpec((ROWS_PER_BLOCK,), lambda i: (i,)),
        pl.BlockSpec((B,), lambda i: (0,)),
        pl.BlockSpec((COPY_BLOCK,), lambda i: (jnp.minimum(i, COPY_LAST),)),
    ],
    out_specs=[
        pl.BlockSpec((ROWS_PER_BLOCK,), lambda i: (i,)),
        pl.BlockSpec((ROWS_PER_BLOCK,), lambda i: (i,)),
        pl.BlockSpec((1, 1), lambda i: (0, 0)),
        pl.BlockSpec((1, 1), lambda i: (0, 0)),
        pl.BlockSpec((COPY_BLOCK,), lambda i: (jnp.minimum(i, COPY_LAST),)),
    ],
    out_shape=[
        jax.ShapeDtypeStruct((B,), jnp.float32),
        jax.ShapeDtypeStruct((B,), jnp.int32),
        jax.ShapeDtypeStruct((1, 1), jnp.float32),
        jax.ShapeDtypeStruct((1, 1), jnp.float32),
        jax.ShapeDtypeStruct((N,), jnp.float32),
    ],
    compiler_params=pltpu.CompilerParams(
        dimension_semantics=("arbitrary",),
    ),
)


@functools.cache
def _make_sc_scatter():
    return functools.partial(
        pl.kernel,
        mesh=plsc.VectorSubcoreMesh(core_axis_name="c", subcore_axis_name="s",
                                    num_cores=1),
        out_type=(),
        scratch_types=[
            pltpu.VMEM((UPD_PER_TILE,), jnp.int32),
            pltpu.VMEM((UPD_PER_TILE,), jnp.int32),
            pltpu.VMEM((UPD_PER_TILE,), jnp.float32),
            pltpu.VMEM((UPD_PER_TILE,), jnp.float32),
            pltpu.VMEM((UPD_PER_TILE,), jnp.float32),
            pltpu.SemaphoreType.DMA,
            pltpu.SemaphoreType.DMA,
            pltpu.SemaphoreType.DMA,
            pltpu.SemaphoreType.DMA,
            pltpu.SemaphoreType.DMA,
        ],
        compiler_params=pltpu.CompilerParams(needs_layout_passes=False),
    )(_sc_scatter_body)


def _sc_scatter_body(out_ref, s_inv_hbm, idx_hbm, xim_hbm, last_hbm,
                     idx_v, last_v, ximf_v, sold_v, vals_v,
                     sem_i, sem_l, sem_x, sem_s, sem_o):
    wid = lax.axis_index("s")
    pos = pl.multiple_of(wid * UPD_PER_TILE, 8)

    cp_i = pltpu.async_copy(idx_hbm.at[pl.ds(pos, UPD_PER_TILE)], idx_v,
                            sem_i)
    cp_l = pltpu.async_copy(last_hbm.at[pl.ds(pos, UPD_PER_TILE)], last_v,
                            sem_l)
    cp_i.wait()
    cp_s = pltpu.async_copy(s_inv_hbm.at[idx_v], sold_v, sem_s)
    cp_l.wait()
    cp_x = pltpu.async_copy(xim_hbm.at[last_v], ximf_v, sem_x)
    cp_s.wait()
    cp_x.wait()

    for k in range(UPD_PER_TILE // 16):
        sl = pl.ds(k * 16, 16)
        vals_v[sl] = RHO * sold_v[sl] + UPD_SCALE * ximf_v[sl]

    pltpu.async_copy(vals_v, out_ref.at[idx_v], sem_o).wait()


def kernel(q_attr_1, q_attr_2, q_rep_1, q_rep_2, feats_idx, s_inv,
           E_attr, E_rep):
    xim, last, sa, sr, buf = _tc_call(q_rep_1, q_rep_2, q_attr_1, q_attr_2,
                                      feats_idx, feats_idx, s_inv)
    ref = jax.new_ref(buf)
    _make_sc_scatter()(ref, s_inv, feats_idx, xim, last)
    s_inv_new = ref[...]
    w = NSQ / (NSQ + 2.0 * B * 100000.0)
    E_attr_new = (1.0 - w) * E_attr + (w / (2.0 * B)) * sa.reshape(1)
    E_rep_new = (1.0 - w) * E_rep + (w / (2.0 * B * W)) * sr.reshape(1)
    return (s_inv_new, E_attr_new, E_rep_new)
